# baseline (speedup 1.0000x reference)
import functools

import jax
import jax.numpy as jnp
from jax import lax
from jax.experimental import pallas as pl
from jax.experimental.pallas import tpu as pltpu
from jax.experimental.pallas import tpu_sc as plsc

NC, NS, L = 2, 16, 16
NW = NC * NS
CB = 128

_MESH = plsc.VectorSubcoreMesh(core_axis_name="c", subcore_axis_name="s")

def _rsqrt16(x):
    i = plsc.bitcast(x, jnp.int32)
    i = 0x5F3759DF - lax.shift_right_arithmetic(i, 1)
    y = plsc.bitcast(i, jnp.float32)
    for _ in range(3):
        y = y * (1.5 - 0.5 * x * y * y)
    return y


@functools.cache
def _make_sh_kernel(n_nodes, n_edge):
    ew = n_edge // NW
    cs = 400
    nch = ew // cs
    assert nch * cs == ew
    sqrt3 = 3.0 ** 0.5

    @functools.partial(
        pl.kernel,
        out_type=jax.ShapeDtypeStruct((n_edge, 8), jnp.float32),
        mesh=_MESH,
        compiler_params=pltpu.CompilerParams(needs_layout_passes=False),
        scratch_types=[
            pltpu.VMEM((3 * n_nodes,), jnp.float32),
            pltpu.VMEM((ew,), jnp.int32),
            pltpu.VMEM((ew,), jnp.int32),
            pltpu.VMEM((cs, 8), jnp.float32),
        ],
    )
    def sh_kernel(pos_hbm, send_hbm, recv_hbm, sh_hbm, pos_v, sidx_v, ridx_v,
                  out_v):
        wid = lax.axis_index("s") * NC + lax.axis_index("c")
        ebase = wid * ew
        pltpu.sync_copy(pos_hbm, pos_v)
        pltpu.sync_copy(send_hbm.at[pl.ds(ebase, ew)], sidx_v)
        pltpu.sync_copy(recv_hbm.at[pl.ds(ebase, ew)], ridx_v)
        lane = lax.iota(jnp.int32, L)
        ones = jnp.full((L,), 1.0, jnp.float32)
        zeros = jnp.zeros((L,), jnp.float32)

        def prefill(g, _):
            rows = lane + g * L
            plsc.store_scatter(out_v, [rows, lane * 0], ones)
            for col in (4, 5, 6, 7):
                plsc.store_scatter(out_v, [rows, lane * 0 + col], zeros)
            return _

        lax.fori_loop(0, cs // L, prefill, None)

        def chunk(k, _):
            def group(g, _):
                off = k * cs + g * L
                si = sidx_v[pl.ds(off, L)]
                ri = ridx_v[pl.ds(off, L)]
                rx = plsc.load_gather(pos_v, [si]) - plsc.load_gather(pos_v, [ri])
                ry = (plsc.load_gather(pos_v, [si + n_nodes])
                      - plsc.load_gather(pos_v, [ri + n_nodes]))
                rz = (plsc.load_gather(pos_v, [si + 2 * n_nodes])
                      - plsc.load_gather(pos_v, [ri + 2 * n_nodes]))
                n2 = rx * rx + ry * ry + rz * rz
                nrm = n2 * _rsqrt16(n2)
                w = sqrt3 / (nrm + 1e-8)
                rows = lane + g * L
                plsc.store_scatter(out_v, [rows, lane * 0 + 1], rx * w)
                plsc.store_scatter(out_v, [rows, lane * 0 + 2], ry * w)
                plsc.store_scatter(out_v, [rows, lane * 0 + 3], rz * w)
                return _

            lax.fori_loop(0, cs // L, group, None)
            pltpu.sync_copy(out_v, sh_hbm.at[pl.ds(ebase + k * cs, cs)])
            return _

        lax.fori_loop(0, nch, chunk, None)

    return sh_kernel


@functools.cache
def _make_scatter_kernel(n_nodes, n_edge, width):
    ew = n_edge // NW
    cb = 80 if ew % 80 == 0 else 40
    nch = ew // cb
    assert nch * cb == ew
    rows_per_sub = -(-n_nodes // (NS * 8)) * 8
    n_pad = rows_per_sub * NS

    @functools.partial(
        pl.kernel,
        out_type=jax.ShapeDtypeStruct((NC, n_pad, width), jnp.float32),
        mesh=_MESH,
        compiler_params=pltpu.CompilerParams(needs_layout_passes=False),
        scratch_types=[
            pltpu.VMEM_SHARED((n_pad, width), jnp.float32),
            pltpu.VMEM((cb, width), jnp.float32),
            pltpu.VMEM((cb, width), jnp.float32),
            pltpu.VMEM((cb,), jnp.int32),
            pltpu.VMEM((cb,), jnp.int32),
            pltpu.SemaphoreType.DMA,
            pltpu.SemaphoreType.DMA,
        ],
    )
    def scatter_kernel(vals_hbm, recv_hbm, zeros_hbm, out_hbm, acc, rows_v0,
                       rows_v1, idx_v0, idx_v1, sem0, sem1):
        cid = lax.axis_index("c")
        sid = lax.axis_index("s")
        wid = sid * NC + cid
        ebase = wid * ew
        rows_v = (rows_v0, rows_v1)
        idx_v = (idx_v0, idx_v1)
        sems = (sem0, sem1)
        pltpu.sync_copy(zeros_hbm, acc.at[pl.ds(sid * rows_per_sub, rows_per_sub)])
        plsc.subcore_barrier()

        def fire(j, x):
            base = ebase + j * cb
            pltpu.async_copy(recv_hbm.at[pl.ds(base, cb)], idx_v[x], sems[x])
            pltpu.async_copy(vals_hbm.at[pl.ds(base, cb)], rows_v[x], sems[x])

        def drain(x):
            pltpu.make_async_copy(recv_hbm.at[pl.ds(0, cb)], idx_v[x], sems[x]).wait()
            pltpu.make_async_copy(vals_hbm.at[pl.ds(0, cb)], rows_v[x], sems[x]).wait()

        def scat(x):
            pltpu.sync_copy(rows_v[x], acc.at[idx_v[x]], add=True)

        fire(0, 0)

        def pair(i, _):
            j0 = 2 * i

            @pl.when(j0 + 1 < nch)
            def _():
                fire(j0 + 1, 1)
            drain(0)
            scat(0)

            @pl.when(j0 + 2 < nch)
            def _():
                fire(j0 + 2, 0)

            @pl.when(j0 + 1 < nch)
            def _():
                drain(1)
                scat(1)
            return _

        lax.fori_loop(0, (nch + 1) // 2, pair, None)
        plsc.subcore_barrier()
        sl = pl.ds(sid * rows_per_sub, rows_per_sub)
        pltpu.sync_copy(acc.at[sl], out_hbm.at[cid, sl])

    return scatter_kernel


@functools.cache
def _make_edge1_kernel(n_nodes, n_edge, hdim):
    ew = n_edge // NW
    cb = 80 if ew % 80 == 0 else 40
    nch = ew // cb
    assert nch * cb == ew
    tw = 2 * hdim

    @functools.partial(
        pl.kernel,
        out_type=jax.ShapeDtypeStruct((n_edge, hdim // 2), jnp.int32),
        mesh=_MESH,
        compiler_params=pltpu.CompilerParams(needs_layout_passes=False),
        scratch_types=[
            pltpu.VMEM((ew,), jnp.int32),
            pltpu.VMEM((ew,), jnp.int32),
            pltpu.VMEM((cb, tw), jnp.int32),
            pltpu.VMEM((cb, tw), jnp.int32),
            pltpu.VMEM((cb, tw), jnp.int32),
            pltpu.VMEM((cb, tw), jnp.int32),
            pltpu.VMEM((cb, 8), jnp.float32),
            pltpu.VMEM((cb, 8), jnp.float32),
            pltpu.VMEM((cb, hdim // 2), jnp.int32),
            pltpu.VMEM((cb, hdim // 2), jnp.int32),
            pltpu.SemaphoreType.DMA,
            pltpu.SemaphoreType.DMA,
            pltpu.SemaphoreType.DMA,
            pltpu.SemaphoreType.DMA,
        ],
    )
    def edge1_kernel(top_hbm, bot_hbm, send_hbm, recv_hbm, sh_hbm, out_hbm,
                     sidx_all, ridx_all, a_v0, b_v0, a_v1, b_v1, sh_v0, sh_v1,
                     o_v0, o_v1, sem0, sem1, semo0, semo1):
        wid = lax.axis_index("s") * NC + lax.axis_index("c")
        ebase = wid * ew
        a_v = (a_v0, a_v1)
        b_v = (b_v0, b_v1)
        sh_v = (sh_v0, sh_v1)
        o_v = (o_v0, o_v1)
        sems = (sem0, sem1)
        semo = (semo0, semo1)
        pltpu.sync_copy(send_hbm.at[pl.ds(ebase, ew)], sidx_all)
        pltpu.sync_copy(recv_hbm.at[pl.ds(ebase, ew)], ridx_all)

        def fire(j, x):
            sl = pl.ds(j * cb, cb)
            base = ebase + j * cb
            pltpu.async_copy(top_hbm.at[sidx_all.at[sl]], a_v[x], sems[x])
            pltpu.async_copy(bot_hbm.at[ridx_all.at[sl]], b_v[x], sems[x])
            pltpu.async_copy(sh_hbm.at[pl.ds(base, cb)], sh_v[x], sems[x])

        def drain(x):
            pltpu.make_async_copy(top_hbm.at[pl.ds(0, cb)], a_v[x], sems[x]).wait()
            pltpu.make_async_copy(bot_hbm.at[pl.ds(0, cb)], b_v[x], sems[x]).wait()
            pltpu.make_async_copy(sh_hbm.at[pl.ds(0, cb)], sh_v[x], sems[x]).wait()

        def outfire(j, x):
            base = ebase + j * cb
            pltpu.async_copy(o_v[x], out_hbm.at[pl.ds(base, cb)], semo[x])

        def outdrain(x):
            pltpu.make_async_copy(o_v[x], out_hbm.at[pl.ds(0, cb)],
                                  semo[x]).wait()

        def compute(j, x):
            av, bv, shv, ov = a_v[x], b_v[x], sh_v[x], o_v[x]

            def edge(e, _):
                zero16 = jnp.zeros((L,), jnp.int32)
                shb = [plsc.load_gather(shv, [zero16 + e, zero16 + a])
                       for a in range(4)]
                accs = [None] * (hdim // L)
                for m in range(tw // L):
                    a_lo = m // (hdim // L)
                    a_hi = 2 + a_lo
                    q = m % (hdim // L)
                    sl = pl.ds(m * L, L)
                    lo = hi = None
                    for wv in (av[e, sl], bv[e, sl]):
                        wl = plsc.bitcast(lax.shift_left(wv, 16), jnp.float32)
                        wh = plsc.bitcast(wv & (-65536), jnp.float32)
                        lo = wl if lo is None else lo + wl
                        hi = wh if hi is None else hi + wh
                    v = shb[a_lo] * lo + shb[a_hi] * hi
                    accs[q] = v if accs[q] is None else accs[q] + v
                nqh = hdim // (2 * L)
                for q in range(nqh):
                    lo_b = lax.shift_right_logical(
                        plsc.bitcast(accs[q], jnp.int32), 16)
                    hi_b = plsc.bitcast(accs[q + nqh], jnp.int32) & (-65536)
                    ov[e, pl.ds(q * L, L)] = lo_b | hi_b
                return _

            lax.fori_loop(0, cb, edge, None)

        fire(0, 0)

        def pair(i, _):
            j0 = 2 * i

            @pl.when(j0 + 1 < nch)
            def _():
                fire(j0 + 1, 1)
            drain(0)

            @pl.when(j0 >= 2)
            def _():
                outdrain(0)
            compute(j0, 0)
            outfire(j0, 0)

            @pl.when(j0 + 2 < nch)
            def _():
                fire(j0 + 2, 0)

            @pl.when(j0 + 1 < nch)
            def _():
                drain(1)

                @pl.when(j0 >= 1)
                def _():
                    outdrain(1)
                compute(j0 + 1, 1)
                outfire(j0 + 1, 1)
            return _

        lax.fori_loop(0, (nch + 1) // 2, pair, None)
        outdrain(0)
        outdrain(1)

    return edge1_kernel


def _node_tables(h, w0t, w0b):
    n, d = h.shape
    td = w0t.shape[1]
    bn = 400

    def pack_words(z):
        zb = z.astype(jnp.bfloat16)
        half = zb.shape[1] // 2
        lo = lax.bitcast_convert_type(zb[:, :half], jnp.uint16)
        hi = lax.bitcast_convert_type(zb[:, half:], jnp.uint16)
        return lo.astype(jnp.int32) | lax.shift_left(hi.astype(jnp.int32), 16)

    def body(h_ref, wt_ref, wb_ref, top_ref, bot_ref):
        hb = h_ref[...]
        top_ref[...] = pack_words(jnp.dot(hb, wt_ref[...],
                                          preferred_element_type=jnp.float32))
        bot_ref[...] = pack_words(jnp.dot(hb, wb_ref[...],
                                          preferred_element_type=jnp.float32))

    return pl.pallas_call(
        body,
        grid=(n // bn,),
        in_specs=[
            pl.BlockSpec((bn, d), lambda i: (i, 0)),
            pl.BlockSpec((d, td), lambda i: (0, 0)),
            pl.BlockSpec((d, td), lambda i: (0, 0)),
        ],
        out_specs=[
            pl.BlockSpec((bn, td // 2), lambda i: (i, 0)),
            pl.BlockSpec((bn, td // 2), lambda i: (i, 0)),
        ],
        out_shape=[
            jax.ShapeDtypeStruct((n, td // 2), jnp.int32),
            jax.ShapeDtypeStruct((n, td // 2), jnp.int32),
        ],
    )(h, w0t, w0b)


def _edge_layer2(t1, sh8, w1r, b0, b1):
    e, hdim = t1.shape[0], 2 * t1.shape[1]
    be = 4000

    def body(t_ref, sh_ref, w_ref, b0_ref, b1_ref, out_ref):
        w = t_ref[...]
        t_lo = lax.bitcast_convert_type(lax.shift_left(w, 16), jnp.float32)
        t_hi = lax.bitcast_convert_type(w & (-65536), jnp.float32)
        g_lo = jax.nn.gelu(t_lo + b0_ref[:, :hdim // 2]).astype(jnp.bfloat16)
        g_hi = jax.nn.gelu(t_hi + b0_ref[:, hdim // 2:]).astype(jnp.bfloat16)
        u = (jnp.dot(g_lo, w_ref[:hdim // 2],
                     preferred_element_type=jnp.float32)
             + jnp.dot(g_hi, w_ref[hdim // 2:],
                       preferred_element_type=jnp.float32))
        acc = jnp.broadcast_to(b1_ref[...], (be, hdim))
        for a in range(4):
            acc = acc + sh_ref[:, a:a + 1] * u[:, a * hdim:(a + 1) * hdim]
        out_ref[...] = acc

    return pl.pallas_call(
        body,
        grid=(e // be,),
        in_specs=[
            pl.BlockSpec((be, hdim // 2), lambda i: (i, 0)),
            pl.BlockSpec((be, 8), lambda i: (i, 0)),
            pl.BlockSpec((hdim, 4 * hdim), lambda i: (0, 0)),
            pl.BlockSpec((1, hdim), lambda i: (0, 0)),
            pl.BlockSpec((1, hdim), lambda i: (0, 0)),
        ],
        out_specs=pl.BlockSpec((be, hdim), lambda i: (i, 0)),
        out_shape=jax.ShapeDtypeStruct((e, hdim), jnp.float32),
    )(t1, sh8, w1r, b0, b1)


def _node_update(h, ms_list, a_parts, w0r, w1r, wres, bn0, bn1, n_edge):
    n, d = h.shape
    hdim = ms_list[0].shape[2]
    n_ms = len(ms_list)
    na = w0r.shape[1] // d
    bn = 400
    ca = 17
    inv_em1 = 1.0 / (n_edge - 1)

    def mix(hb, mcols, wref, acc):
        hb16 = hb.astype(jnp.bfloat16)
        for c in range(na // ca):
            z = jnp.dot(hb16, wref[:, c * ca * d:(c + 1) * ca * d],
                        preferred_element_type=jnp.float32)
            for a in range(ca):
                k = c * ca + a
                acc = acc + mcols[:, k:k + 1] * z[:, a * d:(a + 1) * d]
        return acc

    def body(h_ref, *refs):
        ms_refs = refs[:2 * n_ms]
        ap0_ref, ap1_ref, w0_ref, w1_ref, wr_ref, b0_ref, b1_ref, out_ref = \
            refs[2 * n_ms:]
        hb = h_ref[...]
        a8 = ap0_ref[0] + ap1_ref[0]
        cnt = jnp.maximum(a8[:, 0:1], 1.0)
        msum = ms_refs[0][0]
        for r in ms_refs[1:]:
            msum = msum + r[0]
        magg = msum / cnt * inv_em1
        a_n = a8[:, :4] * inv_em1
        mcols = jnp.concatenate([magg, a_n], axis=1)
        acc = jnp.broadcast_to(b0_ref[...], (bn, d))
        g = jax.nn.gelu(mix(hb, mcols, w0_ref, acc))
        acc2 = jnp.broadcast_to(b1_ref[...], (bn, d))
        acc2 = mix(g, mcols, w1_ref, acc2)
        out_ref[...] = acc2 + jnp.dot(hb, wr_ref[...],
                                      preferred_element_type=jnp.float32)

    return pl.pallas_call(
        body,
        grid=(n // bn,),
        in_specs=[
            pl.BlockSpec((bn, d), lambda i: (i, 0)),
        ] + [
            spec for _ in range(n_ms) for spec in (
                pl.BlockSpec((1, bn, hdim), lambda i: (0, i, 0)),
                pl.BlockSpec((1, bn, hdim), lambda i: (1, i, 0)),
            )
        ] + [
            pl.BlockSpec((1, bn, 8), lambda i: (0, i, 0)),
            pl.BlockSpec((1, bn, 8), lambda i: (1, i, 0)),
            pl.BlockSpec((d, na * d), lambda i: (0, 0)),
            pl.BlockSpec((d, na * d), lambda i: (0, 0)),
            pl.BlockSpec((d, d), lambda i: (0, 0)),
            pl.BlockSpec((1, d), lambda i: (0, 0)),
            pl.BlockSpec((1, d), lambda i: (0, 0)),
        ],
        out_specs=pl.BlockSpec((bn, d), lambda i: (i, 0)),
        out_shape=jax.ShapeDtypeStruct((n, d), jnp.float32),
    )(h, *[m for ms in ms_list for m in (ms, ms)],
      a_parts, a_parts, w0r, w1r, wres, bn0, bn1)


def kernel(nodes, senders, receivers, W_e0, b_e0, W_e1, b_e1, W_n0, b_n0,
           W_n1, b_n1, W_res):
    n, d = nodes.shape
    e = senders.shape[0]
    s_steps, _, a_dim, hdim = W_e0.shape

    pos3 = nodes[:, :3].T.reshape(-1)
    sh8 = _make_sh_kernel(n, e)(pos3, senders, receivers)

    rps = -(-n // (NS * 8)) * 8
    zeros8 = jnp.zeros((rps, 8), jnp.float32)
    zeros64 = jnp.zeros((rps, hdim), jnp.float32)
    a_parts = _make_scatter_kernel(n, e, 8)(sh8, receivers, zeros8)

    bounds = (0, e // 5, e // 5 + 2 * e // 5, e)
    pieces = [(senders[lo:hi], receivers[lo:hi], sh8[lo:hi],
               _make_edge1_kernel(n, hi - lo, hdim),
               _make_scatter_kernel(n, hi - lo, hdim))
              for lo, hi in zip(bounds[:-1], bounds[1:])]

    h = nodes
    for s in range(s_steps):
        w0 = W_e0[s].reshape(2 * d, a_dim * hdim)
        w1 = W_e1[s].reshape(hdim, a_dim * hdim).astype(jnp.bfloat16)
        b0 = b_e0[s].reshape(1, hdim)
        b1 = b_e1[s].reshape(1, hdim)
        top, bot = _node_tables(h, w0[:d], w0[d:])
        t1s = [edge1(top, bot, sp, rp, shp)
               for sp, rp, shp, edge1, _ in pieces]
        ms_list = []
        for t1, (sp, rp, shp, _, scat) in zip(t1s, pieces):
            t2 = _edge_layer2(t1, shp, w1, b0, b1)
            ms_list.append(scat(t2, rp, zeros64))
        h = _node_update(h, ms_list, a_parts,
                         W_n0[s].reshape(d, -1).astype(jnp.bfloat16),
                         W_n1[s].reshape(d, -1).astype(jnp.bfloat16),
                         W_res[s], b_n0[s].reshape(1, d), b_n1[s].reshape(1, d),
                         e)
    return h


# speedup vs baseline: 1.0081x; 1.0081x over previous
import functools

import jax
import jax.numpy as jnp
from jax import lax
from jax.experimental import pallas as pl
from jax.experimental.pallas import tpu as pltpu
from jax.experimental.pallas import tpu_sc as plsc

NC, NS, L = 2, 16, 16
NW = NC * NS
CB = 128

_MESH = plsc.VectorSubcoreMesh(core_axis_name="c", subcore_axis_name="s")

def _rsqrt16(x):
    i = plsc.bitcast(x, jnp.int32)
    i = 0x5F3759DF - lax.shift_right_arithmetic(i, 1)
    y = plsc.bitcast(i, jnp.float32)
    for _ in range(3):
        y = y * (1.5 - 0.5 * x * y * y)
    return y


@functools.cache
def _make_sh_kernel(n_nodes, n_edge):
    ew = n_edge // NW
    cs = 400
    nch = ew // cs
    assert nch * cs == ew
    sqrt3 = 3.0 ** 0.5

    @functools.partial(
        pl.kernel,
        out_type=jax.ShapeDtypeStruct((n_edge, 8), jnp.float32),
        mesh=_MESH,
        compiler_params=pltpu.CompilerParams(needs_layout_passes=False),
        scratch_types=[
            pltpu.VMEM((3 * n_nodes,), jnp.float32),
            pltpu.VMEM((ew,), jnp.int32),
            pltpu.VMEM((ew,), jnp.int32),
            pltpu.VMEM((cs, 8), jnp.float32),
        ],
    )
    def sh_kernel(pos_hbm, send_hbm, recv_hbm, sh_hbm, pos_v, sidx_v, ridx_v,
                  out_v):
        wid = lax.axis_index("s") * NC + lax.axis_index("c")
        ebase = wid * ew
        pltpu.sync_copy(pos_hbm, pos_v)
        pltpu.sync_copy(send_hbm.at[pl.ds(ebase, ew)], sidx_v)
        pltpu.sync_copy(recv_hbm.at[pl.ds(ebase, ew)], ridx_v)
        lane = lax.iota(jnp.int32, L)
        ones = jnp.full((L,), 1.0, jnp.float32)
        zeros = jnp.zeros((L,), jnp.float32)

        def prefill(g, _):
            rows = lane + g * L
            plsc.store_scatter(out_v, [rows, lane * 0], ones)
            for col in (4, 5, 6, 7):
                plsc.store_scatter(out_v, [rows, lane * 0 + col], zeros)
            return _

        lax.fori_loop(0, cs // L, prefill, None)

        def chunk(k, _):
            def group(g, _):
                off = k * cs + g * L
                si = sidx_v[pl.ds(off, L)]
                ri = ridx_v[pl.ds(off, L)]
                rx = plsc.load_gather(pos_v, [si]) - plsc.load_gather(pos_v, [ri])
                ry = (plsc.load_gather(pos_v, [si + n_nodes])
                      - plsc.load_gather(pos_v, [ri + n_nodes]))
                rz = (plsc.load_gather(pos_v, [si + 2 * n_nodes])
                      - plsc.load_gather(pos_v, [ri + 2 * n_nodes]))
                n2 = rx * rx + ry * ry + rz * rz
                nrm = n2 * _rsqrt16(n2)
                w = sqrt3 / (nrm + 1e-8)
                rows = lane + g * L
                plsc.store_scatter(out_v, [rows, lane * 0 + 1], rx * w)
                plsc.store_scatter(out_v, [rows, lane * 0 + 2], ry * w)
                plsc.store_scatter(out_v, [rows, lane * 0 + 3], rz * w)
                return _

            lax.fori_loop(0, cs // L, group, None)
            pltpu.sync_copy(out_v, sh_hbm.at[pl.ds(ebase + k * cs, cs)])
            return _

        lax.fori_loop(0, nch, chunk, None)

    return sh_kernel


@functools.cache
def _make_scatter_kernel(n_nodes, n_edge, width):
    ew = n_edge // NW
    cb = 80 if ew % 80 == 0 else 40
    nch = ew // cb
    assert nch * cb == ew
    rows_per_sub = -(-n_nodes // (NS * 8)) * 8
    n_pad = rows_per_sub * NS

    @functools.partial(
        pl.kernel,
        out_type=jax.ShapeDtypeStruct((NC, n_pad, width), jnp.float32),
        mesh=_MESH,
        compiler_params=pltpu.CompilerParams(needs_layout_passes=False),
        scratch_types=[
            pltpu.VMEM_SHARED((n_pad, width), jnp.float32),
            pltpu.VMEM((cb, width), jnp.float32),
            pltpu.VMEM((cb, width), jnp.float32),
            pltpu.VMEM((cb,), jnp.int32),
            pltpu.VMEM((cb,), jnp.int32),
            pltpu.SemaphoreType.DMA,
            pltpu.SemaphoreType.DMA,
        ],
    )
    def scatter_kernel(vals_hbm, recv_hbm, zeros_hbm, out_hbm, acc, rows_v0,
                       rows_v1, idx_v0, idx_v1, sem0, sem1):
        cid = lax.axis_index("c")
        sid = lax.axis_index("s")
        wid = sid * NC + cid
        ebase = wid * ew
        rows_v = (rows_v0, rows_v1)
        idx_v = (idx_v0, idx_v1)
        sems = (sem0, sem1)
        pltpu.sync_copy(zeros_hbm, acc.at[pl.ds(sid * rows_per_sub, rows_per_sub)])
        plsc.subcore_barrier()

        def fire(j, x):
            base = ebase + j * cb
            pltpu.async_copy(recv_hbm.at[pl.ds(base, cb)], idx_v[x], sems[x])
            pltpu.async_copy(vals_hbm.at[pl.ds(base, cb)], rows_v[x], sems[x])

        def drain(x):
            pltpu.make_async_copy(recv_hbm.at[pl.ds(0, cb)], idx_v[x], sems[x]).wait()
            pltpu.make_async_copy(vals_hbm.at[pl.ds(0, cb)], rows_v[x], sems[x]).wait()

        def scat(x):
            pltpu.sync_copy(rows_v[x], acc.at[idx_v[x]], add=True)

        fire(0, 0)

        def pair(i, _):
            j0 = 2 * i

            @pl.when(j0 + 1 < nch)
            def _():
                fire(j0 + 1, 1)
            drain(0)
            scat(0)

            @pl.when(j0 + 2 < nch)
            def _():
                fire(j0 + 2, 0)

            @pl.when(j0 + 1 < nch)
            def _():
                drain(1)
                scat(1)
            return _

        lax.fori_loop(0, (nch + 1) // 2, pair, None)
        plsc.subcore_barrier()
        sl = pl.ds(sid * rows_per_sub, rows_per_sub)
        pltpu.sync_copy(acc.at[sl], out_hbm.at[cid, sl])

    return scatter_kernel


@functools.cache
def _make_edge1_kernel(n_nodes, n_edge, hdim):
    ew = n_edge // NW
    cb = 80 if ew % 80 == 0 else 40
    nch = ew // cb
    assert nch * cb == ew
    tw = 2 * hdim

    @functools.partial(
        pl.kernel,
        out_type=jax.ShapeDtypeStruct((n_edge, hdim), jnp.float32),
        mesh=_MESH,
        compiler_params=pltpu.CompilerParams(needs_layout_passes=False),
        scratch_types=[
            pltpu.VMEM((ew,), jnp.int32),
            pltpu.VMEM((ew,), jnp.int32),
            pltpu.VMEM((cb, tw), jnp.int32),
            pltpu.VMEM((cb, tw), jnp.int32),
            pltpu.VMEM((cb, tw), jnp.int32),
            pltpu.VMEM((cb, tw), jnp.int32),
            pltpu.VMEM((cb, 8), jnp.float32),
            pltpu.VMEM((cb, 8), jnp.float32),
            pltpu.VMEM((cb, hdim), jnp.float32),
            pltpu.VMEM((cb, hdim), jnp.float32),
            pltpu.SemaphoreType.DMA,
            pltpu.SemaphoreType.DMA,
            pltpu.SemaphoreType.DMA,
            pltpu.SemaphoreType.DMA,
        ],
    )
    def edge1_kernel(top_hbm, bot_hbm, send_hbm, recv_hbm, sh_hbm, out_hbm,
                     sidx_all, ridx_all, a_v0, b_v0, a_v1, b_v1, sh_v0, sh_v1,
                     o_v0, o_v1, sem0, sem1, semo0, semo1):
        wid = lax.axis_index("s") * NC + lax.axis_index("c")
        ebase = wid * ew
        a_v = (a_v0, a_v1)
        b_v = (b_v0, b_v1)
        sh_v = (sh_v0, sh_v1)
        o_v = (o_v0, o_v1)
        sems = (sem0, sem1)
        semo = (semo0, semo1)
        pltpu.sync_copy(send_hbm.at[pl.ds(ebase, ew)], sidx_all)
        pltpu.sync_copy(recv_hbm.at[pl.ds(ebase, ew)], ridx_all)

        def fire(j, x):
            sl = pl.ds(j * cb, cb)
            base = ebase + j * cb
            pltpu.async_copy(top_hbm.at[sidx_all.at[sl]], a_v[x], sems[x])
            pltpu.async_copy(bot_hbm.at[ridx_all.at[sl]], b_v[x], sems[x])
            pltpu.async_copy(sh_hbm.at[pl.ds(base, cb)], sh_v[x], sems[x])

        def drain(x):
            pltpu.make_async_copy(top_hbm.at[pl.ds(0, cb)], a_v[x], sems[x]).wait()
            pltpu.make_async_copy(bot_hbm.at[pl.ds(0, cb)], b_v[x], sems[x]).wait()
            pltpu.make_async_copy(sh_hbm.at[pl.ds(0, cb)], sh_v[x], sems[x]).wait()

        def outfire(j, x):
            base = ebase + j * cb
            pltpu.async_copy(o_v[x], out_hbm.at[pl.ds(base, cb)], semo[x])

        def outdrain(x):
            pltpu.make_async_copy(o_v[x], out_hbm.at[pl.ds(0, cb)],
                                  semo[x]).wait()

        def compute(j, x):
            av, bv, shv, ov = a_v[x], b_v[x], sh_v[x], o_v[x]

            def edge(e, _):
                zero16 = jnp.zeros((L,), jnp.int32)
                shb = [plsc.load_gather(shv, [zero16 + e, zero16 + a])
                       for a in range(4)]
                accs = [None] * (hdim // L)
                for m in range(tw // L):
                    a_lo = m // (hdim // L)
                    a_hi = 2 + a_lo
                    q = m % (hdim // L)
                    sl = pl.ds(m * L, L)
                    lo = hi = None
                    for wv in (av[e, sl], bv[e, sl]):
                        wl = plsc.bitcast(lax.shift_left(wv, 16), jnp.float32)
                        wh = plsc.bitcast(wv & (-65536), jnp.float32)
                        lo = wl if lo is None else lo + wl
                        hi = wh if hi is None else hi + wh
                    v = shb[a_lo] * lo + shb[a_hi] * hi
                    accs[q] = v if accs[q] is None else accs[q] + v
                for q in range(hdim // L):
                    ov[e, pl.ds(q * L, L)] = accs[q]
                return _

            lax.fori_loop(0, cb, edge, None)

        fire(0, 0)

        def pair(i, _):
            j0 = 2 * i

            @pl.when(j0 + 1 < nch)
            def _():
                fire(j0 + 1, 1)
            drain(0)

            @pl.when(j0 >= 2)
            def _():
                outdrain(0)
            compute(j0, 0)
            outfire(j0, 0)

            @pl.when(j0 + 2 < nch)
            def _():
                fire(j0 + 2, 0)

            @pl.when(j0 + 1 < nch)
            def _():
                drain(1)

                @pl.when(j0 >= 1)
                def _():
                    outdrain(1)
                compute(j0 + 1, 1)
                outfire(j0 + 1, 1)
            return _

        lax.fori_loop(0, (nch + 1) // 2, pair, None)
        outdrain(0)
        outdrain(1)

    return edge1_kernel


def _node_tables(h, w0t, w0b):
    n, d = h.shape
    td = w0t.shape[1]
    bn = 400

    def pack_words(z):
        zb = z.astype(jnp.bfloat16)
        half = zb.shape[1] // 2
        lo = lax.bitcast_convert_type(zb[:, :half], jnp.uint16)
        hi = lax.bitcast_convert_type(zb[:, half:], jnp.uint16)
        return lo.astype(jnp.int32) | lax.shift_left(hi.astype(jnp.int32), 16)

    def body(h_ref, wt_ref, wb_ref, top_ref, bot_ref):
        hb = h_ref[...]
        top_ref[...] = pack_words(jnp.dot(hb, wt_ref[...],
                                          preferred_element_type=jnp.float32))
        bot_ref[...] = pack_words(jnp.dot(hb, wb_ref[...],
                                          preferred_element_type=jnp.float32))

    return pl.pallas_call(
        body,
        grid=(n // bn,),
        in_specs=[
            pl.BlockSpec((bn, d), lambda i: (i, 0)),
            pl.BlockSpec((d, td), lambda i: (0, 0)),
            pl.BlockSpec((d, td), lambda i: (0, 0)),
        ],
        out_specs=[
            pl.BlockSpec((bn, td // 2), lambda i: (i, 0)),
            pl.BlockSpec((bn, td // 2), lambda i: (i, 0)),
        ],
        out_shape=[
            jax.ShapeDtypeStruct((n, td // 2), jnp.int32),
            jax.ShapeDtypeStruct((n, td // 2), jnp.int32),
        ],
    )(h, w0t, w0b)


def _edge_layer2(t1, sh8, w1r, b0, b1):
    e, hdim = t1.shape
    be = 4000

    def body(t_ref, sh_ref, w_ref, b0_ref, b1_ref, out_ref):
        g = jax.nn.gelu(t_ref[...] + b0_ref[...])
        u = jnp.dot(g.astype(jnp.bfloat16), w_ref[...],
                    preferred_element_type=jnp.float32)
        acc = jnp.broadcast_to(b1_ref[...], (be, hdim))
        for a in range(4):
            acc = acc + sh_ref[:, a:a + 1] * u[:, a * hdim:(a + 1) * hdim]
        out_ref[...] = acc

    return pl.pallas_call(
        body,
        grid=(e // be,),
        in_specs=[
            pl.BlockSpec((be, hdim), lambda i: (i, 0)),
            pl.BlockSpec((be, 8), lambda i: (i, 0)),
            pl.BlockSpec((hdim, 4 * hdim), lambda i: (0, 0)),
            pl.BlockSpec((1, hdim), lambda i: (0, 0)),
            pl.BlockSpec((1, hdim), lambda i: (0, 0)),
        ],
        out_specs=pl.BlockSpec((be, hdim), lambda i: (i, 0)),
        out_shape=jax.ShapeDtypeStruct((e, hdim), jnp.float32),
    )(t1, sh8, w1r, b0, b1)


def _node_update(h, ms_list, a_parts, w0r, w1r, wres, bn0, bn1, n_edge):
    n, d = h.shape
    hdim = ms_list[0].shape[2]
    n_ms = len(ms_list)
    na = w0r.shape[1] // d
    bn = 400
    ca = 17
    inv_em1 = 1.0 / (n_edge - 1)

    def mix(hb, mcols, wref, acc):
        hb16 = hb.astype(jnp.bfloat16)
        for c in range(na // ca):
            z = jnp.dot(hb16, wref[:, c * ca * d:(c + 1) * ca * d],
                        preferred_element_type=jnp.float32)
            for a in range(ca):
                k = c * ca + a
                acc = acc + mcols[:, k:k + 1] * z[:, a * d:(a + 1) * d]
        return acc

    def body(h_ref, *refs):
        ms_refs = refs[:2 * n_ms]
        ap0_ref, ap1_ref, w0_ref, w1_ref, wr_ref, b0_ref, b1_ref, out_ref = \
            refs[2 * n_ms:]
        hb = h_ref[...]
        a8 = ap0_ref[0] + ap1_ref[0]
        cnt = jnp.maximum(a8[:, 0:1], 1.0)
        msum = ms_refs[0][0]
        for r in ms_refs[1:]:
            msum = msum + r[0]
        magg = msum / cnt * inv_em1
        a_n = a8[:, :4] * inv_em1
        mcols = jnp.concatenate([magg, a_n], axis=1)
        acc = jnp.broadcast_to(b0_ref[...], (bn, d))
        g = jax.nn.gelu(mix(hb, mcols, w0_ref, acc))
        acc2 = jnp.broadcast_to(b1_ref[...], (bn, d))
        acc2 = mix(g, mcols, w1_ref, acc2)
        out_ref[...] = acc2 + jnp.dot(hb, wr_ref[...],
                                      preferred_element_type=jnp.float32)

    return pl.pallas_call(
        body,
        grid=(n // bn,),
        in_specs=[
            pl.BlockSpec((bn, d), lambda i: (i, 0)),
        ] + [
            spec for _ in range(n_ms) for spec in (
                pl.BlockSpec((1, bn, hdim), lambda i: (0, i, 0)),
                pl.BlockSpec((1, bn, hdim), lambda i: (1, i, 0)),
            )
        ] + [
            pl.BlockSpec((1, bn, 8), lambda i: (0, i, 0)),
            pl.BlockSpec((1, bn, 8), lambda i: (1, i, 0)),
            pl.BlockSpec((d, na * d), lambda i: (0, 0)),
            pl.BlockSpec((d, na * d), lambda i: (0, 0)),
            pl.BlockSpec((d, d), lambda i: (0, 0)),
            pl.BlockSpec((1, d), lambda i: (0, 0)),
            pl.BlockSpec((1, d), lambda i: (0, 0)),
        ],
        out_specs=pl.BlockSpec((bn, d), lambda i: (i, 0)),
        out_shape=jax.ShapeDtypeStruct((n, d), jnp.float32),
    )(h, *[m for ms in ms_list for m in (ms, ms)],
      a_parts, a_parts, w0r, w1r, wres, bn0, bn1)


def kernel(nodes, senders, receivers, W_e0, b_e0, W_e1, b_e1, W_n0, b_n0,
           W_n1, b_n1, W_res):
    n, d = nodes.shape
    e = senders.shape[0]
    s_steps, _, a_dim, hdim = W_e0.shape

    pos3 = nodes[:, :3].T.reshape(-1)
    sh8 = _make_sh_kernel(n, e)(pos3, senders, receivers)

    rps = -(-n // (NS * 8)) * 8
    zeros8 = jnp.zeros((rps, 8), jnp.float32)
    zeros64 = jnp.zeros((rps, hdim), jnp.float32)
    a_parts = _make_scatter_kernel(n, e, 8)(sh8, receivers, zeros8)

    bounds = (0, e // 5, e // 5 + 2 * e // 5, e)
    pieces = [(senders[lo:hi], receivers[lo:hi], sh8[lo:hi],
               _make_edge1_kernel(n, hi - lo, hdim),
               _make_scatter_kernel(n, hi - lo, hdim))
              for lo, hi in zip(bounds[:-1], bounds[1:])]

    h = nodes
    for s in range(s_steps):
        w0 = W_e0[s].reshape(2 * d, a_dim * hdim)
        w1 = W_e1[s].reshape(hdim, a_dim * hdim).astype(jnp.bfloat16)
        b0 = b_e0[s].reshape(1, hdim)
        b1 = b_e1[s].reshape(1, hdim)
        top, bot = _node_tables(h, w0[:d], w0[d:])
        t1s = [edge1(top, bot, sp, rp, shp)
               for sp, rp, shp, edge1, _ in pieces]
        ms_list = []
        for t1, (sp, rp, shp, _, scat) in zip(t1s, pieces):
            t2 = _edge_layer2(t1, shp, w1, b0, b1)
            ms_list.append(scat(t2, rp, zeros64))
        h = _node_update(h, ms_list, a_parts,
                         W_n0[s].reshape(d, -1).astype(jnp.bfloat16),
                         W_n1[s].reshape(d, -1).astype(jnp.bfloat16),
                         W_res[s], b_n0[s].reshape(1, d), b_n1[s].reshape(1, d),
                         e)
    return h


# speedup vs baseline: 1.0181x; 1.0099x over previous
import functools

import jax
import jax.numpy as jnp
from jax import lax
from jax.experimental import pallas as pl
from jax.experimental.pallas import tpu as pltpu
from jax.experimental.pallas import tpu_sc as plsc

NC, NS, L = 2, 16, 16
NW = NC * NS
CB = 128

_MESH = plsc.VectorSubcoreMesh(core_axis_name="c", subcore_axis_name="s")

def _rsqrt16(x):
    i = plsc.bitcast(x, jnp.int32)
    i = 0x5F3759DF - lax.shift_right_arithmetic(i, 1)
    y = plsc.bitcast(i, jnp.float32)
    for _ in range(3):
        y = y * (1.5 - 0.5 * x * y * y)
    return y


@functools.cache
def _make_sh_kernel(n_nodes, n_edge):
    ew = n_edge // NW
    cs = 400
    nch = ew // cs
    assert nch * cs == ew
    sqrt3 = 3.0 ** 0.5

    @functools.partial(
        pl.kernel,
        out_type=jax.ShapeDtypeStruct((n_edge, 8), jnp.float32),
        mesh=_MESH,
        compiler_params=pltpu.CompilerParams(needs_layout_passes=False),
        scratch_types=[
            pltpu.VMEM((3 * n_nodes,), jnp.float32),
            pltpu.VMEM((ew,), jnp.int32),
            pltpu.VMEM((ew,), jnp.int32),
            pltpu.VMEM((cs, 8), jnp.float32),
        ],
    )
    def sh_kernel(pos_hbm, send_hbm, recv_hbm, sh_hbm, pos_v, sidx_v, ridx_v,
                  out_v):
        wid = lax.axis_index("s") * NC + lax.axis_index("c")
        ebase = wid * ew
        pltpu.sync_copy(pos_hbm, pos_v)
        pltpu.sync_copy(send_hbm.at[pl.ds(ebase, ew)], sidx_v)
        pltpu.sync_copy(recv_hbm.at[pl.ds(ebase, ew)], ridx_v)
        lane = lax.iota(jnp.int32, L)
        ones = jnp.full((L,), 1.0, jnp.float32)
        zeros = jnp.zeros((L,), jnp.float32)

        def prefill(g, _):
            rows = lane + g * L
            plsc.store_scatter(out_v, [rows, lane * 0], ones)
            for col in (4, 5, 6, 7):
                plsc.store_scatter(out_v, [rows, lane * 0 + col], zeros)
            return _

        lax.fori_loop(0, cs // L, prefill, None)

        def chunk(k, _):
            def group(g, _):
                off = k * cs + g * L
                si = sidx_v[pl.ds(off, L)]
                ri = ridx_v[pl.ds(off, L)]
                rx = plsc.load_gather(pos_v, [si]) - plsc.load_gather(pos_v, [ri])
                ry = (plsc.load_gather(pos_v, [si + n_nodes])
                      - plsc.load_gather(pos_v, [ri + n_nodes]))
                rz = (plsc.load_gather(pos_v, [si + 2 * n_nodes])
                      - plsc.load_gather(pos_v, [ri + 2 * n_nodes]))
                n2 = rx * rx + ry * ry + rz * rz
                nrm = n2 * _rsqrt16(n2)
                w = sqrt3 / (nrm + 1e-8)
                rows = lane + g * L
                plsc.store_scatter(out_v, [rows, lane * 0 + 1], rx * w)
                plsc.store_scatter(out_v, [rows, lane * 0 + 2], ry * w)
                plsc.store_scatter(out_v, [rows, lane * 0 + 3], rz * w)
                return _

            lax.fori_loop(0, cs // L, group, None)
            pltpu.sync_copy(out_v, sh_hbm.at[pl.ds(ebase + k * cs, cs)])
            return _

        lax.fori_loop(0, nch, chunk, None)

    return sh_kernel


@functools.cache
def _make_scatter_kernel(n_nodes, n_edge, width):
    ew = n_edge // NW
    cb = 80 if ew % 80 == 0 else 40
    nch = ew // cb
    assert nch * cb == ew
    rows_per_sub = -(-n_nodes // (NS * 8)) * 8
    n_pad = rows_per_sub * NS

    @functools.partial(
        pl.kernel,
        out_type=jax.ShapeDtypeStruct((NC, n_pad, width), jnp.float32),
        mesh=_MESH,
        compiler_params=pltpu.CompilerParams(needs_layout_passes=False),
        scratch_types=[
            pltpu.VMEM_SHARED((n_pad, width), jnp.float32),
            pltpu.VMEM((cb, width), jnp.float32),
            pltpu.VMEM((cb, width), jnp.float32),
            pltpu.VMEM((cb,), jnp.int32),
            pltpu.VMEM((cb,), jnp.int32),
            pltpu.SemaphoreType.DMA,
            pltpu.SemaphoreType.DMA,
        ],
    )
    def scatter_kernel(vals_hbm, recv_hbm, zeros_hbm, out_hbm, acc, rows_v0,
                       rows_v1, idx_v0, idx_v1, sem0, sem1):
        cid = lax.axis_index("c")
        sid = lax.axis_index("s")
        wid = sid * NC + cid
        ebase = wid * ew
        rows_v = (rows_v0, rows_v1)
        idx_v = (idx_v0, idx_v1)
        sems = (sem0, sem1)
        pltpu.sync_copy(zeros_hbm, acc.at[pl.ds(sid * rows_per_sub, rows_per_sub)])
        plsc.subcore_barrier()

        def fire(j, x):
            base = ebase + j * cb
            pltpu.async_copy(recv_hbm.at[pl.ds(base, cb)], idx_v[x], sems[x])
            pltpu.async_copy(vals_hbm.at[pl.ds(base, cb)], rows_v[x], sems[x])

        def drain(x):
            pltpu.make_async_copy(recv_hbm.at[pl.ds(0, cb)], idx_v[x], sems[x]).wait()
            pltpu.make_async_copy(vals_hbm.at[pl.ds(0, cb)], rows_v[x], sems[x]).wait()

        def scat(x):
            pltpu.sync_copy(rows_v[x], acc.at[idx_v[x]], add=True)

        fire(0, 0)

        def pair(i, _):
            j0 = 2 * i

            @pl.when(j0 + 1 < nch)
            def _():
                fire(j0 + 1, 1)
            drain(0)
            scat(0)

            @pl.when(j0 + 2 < nch)
            def _():
                fire(j0 + 2, 0)

            @pl.when(j0 + 1 < nch)
            def _():
                drain(1)
                scat(1)
            return _

        lax.fori_loop(0, (nch + 1) // 2, pair, None)
        plsc.subcore_barrier()
        sl = pl.ds(sid * rows_per_sub, rows_per_sub)
        pltpu.sync_copy(acc.at[sl], out_hbm.at[cid, sl])

    return scatter_kernel


@functools.cache
def _make_edge1_kernel(n_nodes, n_edge, hdim):
    ew = n_edge // NW
    cb = 80 if ew % 80 == 0 else 40
    nch = ew // cb
    assert nch * cb == ew
    tw = 2 * hdim

    @functools.partial(
        pl.kernel,
        out_type=jax.ShapeDtypeStruct((n_edge, hdim), jnp.float32),
        mesh=_MESH,
        compiler_params=pltpu.CompilerParams(needs_layout_passes=False),
        scratch_types=[
            pltpu.VMEM((ew,), jnp.int32),
            pltpu.VMEM((ew,), jnp.int32),
            pltpu.VMEM((cb, tw), jnp.int32),
            pltpu.VMEM((cb, tw), jnp.int32),
            pltpu.VMEM((cb, tw), jnp.int32),
            pltpu.VMEM((cb, tw), jnp.int32),
            pltpu.VMEM((cb, 8), jnp.float32),
            pltpu.VMEM((cb, 8), jnp.float32),
            pltpu.VMEM((cb, hdim), jnp.float32),
            pltpu.VMEM((cb, hdim), jnp.float32),
            pltpu.SemaphoreType.DMA,
            pltpu.SemaphoreType.DMA,
            pltpu.SemaphoreType.DMA,
            pltpu.SemaphoreType.DMA,
        ],
    )
    def edge1_kernel(top_hbm, bot_hbm, send_hbm, recv_hbm, sh_hbm, out_hbm,
                     sidx_all, ridx_all, a_v0, b_v0, a_v1, b_v1, sh_v0, sh_v1,
                     o_v0, o_v1, sem0, sem1, semo0, semo1):
        wid = lax.axis_index("s") * NC + lax.axis_index("c")
        ebase = wid * ew
        a_v = (a_v0, a_v1)
        b_v = (b_v0, b_v1)
        sh_v = (sh_v0, sh_v1)
        o_v = (o_v0, o_v1)
        sems = (sem0, sem1)
        semo = (semo0, semo1)
        pltpu.sync_copy(send_hbm.at[pl.ds(ebase, ew)], sidx_all)
        pltpu.sync_copy(recv_hbm.at[pl.ds(ebase, ew)], ridx_all)

        def fire(j, x):
            sl = pl.ds(j * cb, cb)
            base = ebase + j * cb
            pltpu.async_copy(top_hbm.at[sidx_all.at[sl]], a_v[x], sems[x])
            pltpu.async_copy(bot_hbm.at[ridx_all.at[sl]], b_v[x], sems[x])
            pltpu.async_copy(sh_hbm.at[pl.ds(base, cb)], sh_v[x], sems[x])

        def drain(x):
            pltpu.make_async_copy(top_hbm.at[pl.ds(0, cb)], a_v[x], sems[x]).wait()
            pltpu.make_async_copy(bot_hbm.at[pl.ds(0, cb)], b_v[x], sems[x]).wait()
            pltpu.make_async_copy(sh_hbm.at[pl.ds(0, cb)], sh_v[x], sems[x]).wait()

        def outfire(j, x):
            base = ebase + j * cb
            pltpu.async_copy(o_v[x], out_hbm.at[pl.ds(base, cb)], semo[x])

        def outdrain(x):
            pltpu.make_async_copy(o_v[x], out_hbm.at[pl.ds(0, cb)],
                                  semo[x]).wait()

        def compute(j, x):
            av, bv, shv, ov = a_v[x], b_v[x], sh_v[x], o_v[x]

            def edge(e, _):
                zero16 = jnp.zeros((L,), jnp.int32)
                shb = [plsc.load_gather(shv, [zero16 + e, zero16 + a])
                       for a in range(4)]
                accs = [None] * (hdim // L)
                for m in range(tw // L):
                    a_lo = m // (hdim // L)
                    a_hi = 2 + a_lo
                    q = m % (hdim // L)
                    sl = pl.ds(m * L, L)
                    lo = hi = None
                    for wv in (av[e, sl], bv[e, sl]):
                        wl = plsc.bitcast(lax.shift_left(wv, 16), jnp.float32)
                        wh = plsc.bitcast(wv & (-65536), jnp.float32)
                        lo = wl if lo is None else lo + wl
                        hi = wh if hi is None else hi + wh
                    v = shb[a_lo] * lo + shb[a_hi] * hi
                    accs[q] = v if accs[q] is None else accs[q] + v
                for q in range(hdim // L):
                    ov[e, pl.ds(q * L, L)] = accs[q]
                return _

            lax.fori_loop(0, cb, edge, None)

        fire(0, 0)

        def pair(i, _):
            j0 = 2 * i

            @pl.when(j0 + 1 < nch)
            def _():
                fire(j0 + 1, 1)
            drain(0)

            @pl.when(j0 >= 2)
            def _():
                outdrain(0)
            compute(j0, 0)
            outfire(j0, 0)

            @pl.when(j0 + 2 < nch)
            def _():
                fire(j0 + 2, 0)

            @pl.when(j0 + 1 < nch)
            def _():
                drain(1)

                @pl.when(j0 >= 1)
                def _():
                    outdrain(1)
                compute(j0 + 1, 1)
                outfire(j0 + 1, 1)
            return _

        lax.fori_loop(0, (nch + 1) // 2, pair, None)
        outdrain(0)
        outdrain(1)

    return edge1_kernel


def _node_tables(h, w0t, w0b):
    n, d = h.shape
    td = w0t.shape[1]
    bn = 400

    def pack_words(z):
        zb = z.astype(jnp.bfloat16)
        half = zb.shape[1] // 2
        lo = lax.bitcast_convert_type(zb[:, :half], jnp.uint16)
        hi = lax.bitcast_convert_type(zb[:, half:], jnp.uint16)
        return lo.astype(jnp.int32) | lax.shift_left(hi.astype(jnp.int32), 16)

    def body(h_ref, wt_ref, wb_ref, top_ref, bot_ref):
        hb = h_ref[...]
        top_ref[...] = pack_words(jnp.dot(hb, wt_ref[...],
                                          preferred_element_type=jnp.float32))
        bot_ref[...] = pack_words(jnp.dot(hb, wb_ref[...],
                                          preferred_element_type=jnp.float32))

    return pl.pallas_call(
        body,
        grid=(n // bn,),
        in_specs=[
            pl.BlockSpec((bn, d), lambda i: (i, 0)),
            pl.BlockSpec((d, td), lambda i: (0, 0)),
            pl.BlockSpec((d, td), lambda i: (0, 0)),
        ],
        out_specs=[
            pl.BlockSpec((bn, td // 2), lambda i: (i, 0)),
            pl.BlockSpec((bn, td // 2), lambda i: (i, 0)),
        ],
        out_shape=[
            jax.ShapeDtypeStruct((n, td // 2), jnp.int32),
            jax.ShapeDtypeStruct((n, td // 2), jnp.int32),
        ],
    )(h, w0t, w0b)


def _edge_layer2(t1, sh8, w1r, b0, b1):
    e, hdim = t1.shape
    be = 8000

    def body(t_ref, sh_ref, w_ref, b0_ref, b1_ref, out_ref):
        g = jax.nn.gelu(t_ref[...] + b0_ref[...])
        u = jnp.dot(g.astype(jnp.bfloat16), w_ref[...],
                    preferred_element_type=jnp.float32)
        acc = jnp.broadcast_to(b1_ref[...], (be, hdim))
        for a in range(4):
            acc = acc + sh_ref[:, a:a + 1] * u[:, a * hdim:(a + 1) * hdim]
        out_ref[...] = acc

    return pl.pallas_call(
        body,
        grid=(e // be,),
        in_specs=[
            pl.BlockSpec((be, hdim), lambda i: (i, 0)),
            pl.BlockSpec((be, 8), lambda i: (i, 0)),
            pl.BlockSpec((hdim, 4 * hdim), lambda i: (0, 0)),
            pl.BlockSpec((1, hdim), lambda i: (0, 0)),
            pl.BlockSpec((1, hdim), lambda i: (0, 0)),
        ],
        out_specs=pl.BlockSpec((be, hdim), lambda i: (i, 0)),
        out_shape=jax.ShapeDtypeStruct((e, hdim), jnp.float32),
    )(t1, sh8, w1r, b0, b1)


def _node_update(h, ms_list, a_parts, w0r, w1r, wres, bn0, bn1, n_edge):
    n, d = h.shape
    hdim = ms_list[0].shape[2]
    n_ms = len(ms_list)
    na = w0r.shape[1] // d
    bn = 1000
    ca = 17
    inv_em1 = 1.0 / (n_edge - 1)

    def mix(hb, mcols, wref, acc):
        hb16 = hb.astype(jnp.bfloat16)
        for c in range(na // ca):
            z = jnp.dot(hb16, wref[:, c * ca * d:(c + 1) * ca * d],
                        preferred_element_type=jnp.float32)
            for a in range(ca):
                k = c * ca + a
                acc = acc + mcols[:, k:k + 1] * z[:, a * d:(a + 1) * d]
        return acc

    def body(h_ref, *refs):
        ms_refs = refs[:2 * n_ms]
        ap0_ref, ap1_ref, w0_ref, w1_ref, wr_ref, b0_ref, b1_ref, out_ref = \
            refs[2 * n_ms:]
        hb = h_ref[...]
        a8 = ap0_ref[0] + ap1_ref[0]
        cnt = jnp.maximum(a8[:, 0:1], 1.0)
        msum = ms_refs[0][0]
        for r in ms_refs[1:]:
            msum = msum + r[0]
        magg = msum / cnt * inv_em1
        a_n = a8[:, :4] * inv_em1
        mcols = jnp.concatenate([magg, a_n], axis=1)
        acc = jnp.broadcast_to(b0_ref[...], (bn, d))
        g = jax.nn.gelu(mix(hb, mcols, w0_ref, acc))
        acc2 = jnp.broadcast_to(b1_ref[...], (bn, d))
        acc2 = mix(g, mcols, w1_ref, acc2)
        out_ref[...] = acc2 + jnp.dot(hb, wr_ref[...],
                                      preferred_element_type=jnp.float32)

    return pl.pallas_call(
        body,
        grid=(n // bn,),
        in_specs=[
            pl.BlockSpec((bn, d), lambda i: (i, 0)),
        ] + [
            spec for _ in range(n_ms) for spec in (
                pl.BlockSpec((1, bn, hdim), lambda i: (0, i, 0)),
                pl.BlockSpec((1, bn, hdim), lambda i: (1, i, 0)),
            )
        ] + [
            pl.BlockSpec((1, bn, 8), lambda i: (0, i, 0)),
            pl.BlockSpec((1, bn, 8), lambda i: (1, i, 0)),
            pl.BlockSpec((d, na * d), lambda i: (0, 0)),
            pl.BlockSpec((d, na * d), lambda i: (0, 0)),
            pl.BlockSpec((d, d), lambda i: (0, 0)),
            pl.BlockSpec((1, d), lambda i: (0, 0)),
            pl.BlockSpec((1, d), lambda i: (0, 0)),
        ],
        out_specs=pl.BlockSpec((bn, d), lambda i: (i, 0)),
        out_shape=jax.ShapeDtypeStruct((n, d), jnp.float32),
    )(h, *[m for ms in ms_list for m in (ms, ms)],
      a_parts, a_parts, w0r, w1r, wres, bn0, bn1)


def kernel(nodes, senders, receivers, W_e0, b_e0, W_e1, b_e1, W_n0, b_n0,
           W_n1, b_n1, W_res):
    n, d = nodes.shape
    e = senders.shape[0]
    s_steps, _, a_dim, hdim = W_e0.shape

    pos3 = nodes[:, :3].T.reshape(-1)
    sh8 = _make_sh_kernel(n, e)(pos3, senders, receivers)

    rps = -(-n // (NS * 8)) * 8
    zeros8 = jnp.zeros((rps, 8), jnp.float32)
    zeros64 = jnp.zeros((rps, hdim), jnp.float32)
    a_parts = _make_scatter_kernel(n, e, 8)(sh8, receivers, zeros8)

    bounds = (0, e // 5, e // 5 + 2 * e // 5, e)
    pieces = [(senders[lo:hi], receivers[lo:hi], sh8[lo:hi],
               _make_edge1_kernel(n, hi - lo, hdim),
               _make_scatter_kernel(n, hi - lo, hdim))
              for lo, hi in zip(bounds[:-1], bounds[1:])]

    h = nodes
    for s in range(s_steps):
        w0 = W_e0[s].reshape(2 * d, a_dim * hdim)
        w1 = W_e1[s].reshape(hdim, a_dim * hdim).astype(jnp.bfloat16)
        b0 = b_e0[s].reshape(1, hdim)
        b1 = b_e1[s].reshape(1, hdim)
        top, bot = _node_tables(h, w0[:d], w0[d:])
        t1s = [edge1(top, bot, sp, rp, shp)
               for sp, rp, shp, edge1, _ in pieces]
        ms_list = []
        for t1, (sp, rp, shp, _, scat) in zip(t1s, pieces):
            t2 = _edge_layer2(t1, shp, w1, b0, b1)
            ms_list.append(scat(t2, rp, zeros64))
        h = _node_update(h, ms_list, a_parts,
                         W_n0[s].reshape(d, -1).astype(jnp.bfloat16),
                         W_n1[s].reshape(d, -1).astype(jnp.bfloat16),
                         W_res[s], b_n0[s].reshape(1, d), b_n1[s].reshape(1, d),
                         e)
    return h
